# Initial kernel scaffold; baseline (speedup 1.0000x reference)
import jax
import jax.numpy as jnp
from jax.experimental import pallas as pl


def kernel(node_feats, edge_index, edge_attr, W1, bW1, a1, ba1, W2, bW2, a2, ba2, Hw1, Hb1, Hw2, Hb2, Hw3, Hb3):
    raise NotImplementedError("write your pallas kernel here")


# speedup vs baseline: 4.2957x; 4.2957x over previous
import functools

import jax
import jax.numpy as jnp
from jax import lax
from jax.experimental import pallas as pl
from jax.experimental.pallas import tpu as pltpu
from jax.experimental.pallas import tpu_sc as plsc

N = 10000
E = 320000
D = 128
DE = 16
H = 64

NC = 2
NS = 16
NW = NC * NS
EPW = E // NW
CK = 128
EPP = 10240
RPW = EPP // CK
E_PAD = NW * EPP
NPT = N // NS
NZF = NPT // CK
NZT = NPT - NZF * CK

_mesh = plsc.VectorSubcoreMesh(
    core_axis_name="c", subcore_axis_name="s", num_cores=NC, num_subcores=NS)


def _worker():
    return lax.axis_index("s") * NC + lax.axis_index("c")


def _sc_scores_body(src_hbm, tgt_hbm, as_hbm, at_hbm,
                    out_hbm, src2_v, tgt2_v, as_v, at_v, sc_v):
    w = _worker()
    pltpu.sync_copy(src_hbm.at[pl.ds(w * RPW, RPW)], src2_v)
    pltpu.sync_copy(tgt_hbm.at[pl.ds(w * RPW, RPW)], tgt2_v)
    pltpu.sync_copy(as_hbm, as_v)
    pltpu.sync_copy(at_hbm, at_v)

    neg = jnp.zeros((16,), jnp.float32) - 3.0e38
    NFR = EPW // CK
    NB = CK // 16

    def _score(j, cc):
        vs = plsc.load_gather(as_v, [src2_v[j, pl.ds(cc * 16, 16)]])
        vt = plsc.load_gather(at_v, [tgt2_v[j, pl.ds(cc * 16, 16)]])
        x = vs + vt
        return jnp.where(x >= 0, x, 0.2 * x)

    @pl.loop(0, NFR)
    def _(r):
        for cc in range(NB):
            sc_v[pl.ds(r * CK + cc * 16, 16)] = _score(r, cc)

    sc_v[pl.ds(NFR * CK, 16)] = _score(NFR, 0)
    for t in range((EPP - EPW) // 16):
        sc_v[pl.ds(EPW + t * 16, 16)] = neg

    pltpu.sync_copy(sc_v, out_hbm.at[pl.ds(w * EPP, EPP)])


_sc_scores = functools.partial(
    pl.kernel,
    out_type=jax.ShapeDtypeStruct((E_PAD,), jnp.float32),
    mesh=_mesh,
    compiler_params=pltpu.CompilerParams(
        needs_layout_passes=False, use_tc_tiling_on_sc=False),
    scratch_types=[
        pltpu.VMEM((RPW, CK), jnp.int32),
        pltpu.VMEM((RPW, CK), jnp.int32),
        pltpu.VMEM((N,), jnp.float32),
        pltpu.VMEM((N,), jnp.float32),
        pltpu.VMEM((EPP,), jnp.float32),
    ],
)(_sc_scores_body)


def _tc_softmax(scores):
    x2 = scores.reshape(E_PAD // 128, 128)

    def body(x_ref, o_ref):
        x = x_ref[...]
        m = jnp.max(x)
        e = jnp.exp(x - m)
        o_ref[...] = e / jnp.sum(e)

    return pl.pallas_call(
        body,
        out_shape=jax.ShapeDtypeStruct(x2.shape, jnp.float32),
    )(x2).reshape(E_PAD)


def _sc_scatter_body(src_hbm, tgt_hbm, wt_hbm, wh_hbm,
                     out_hbm, src2_v, tgt2_v, sc_v,
                     rows0, rows1, acc_sh, sem0, sem1):
    c = lax.axis_index("c")
    s = lax.axis_index("s")
    w = s * NC + c
    pltpu.sync_copy(src_hbm.at[pl.ds(w * RPW, RPW)], src2_v)
    pltpu.sync_copy(tgt_hbm.at[pl.ds(w * RPW, RPW)], tgt2_v)
    pltpu.sync_copy(wt_hbm.at[pl.ds(w * EPP, EPP)], sc_v)

    z16 = jnp.zeros((16,), jnp.float32)

    @pl.loop(0, CK)
    def _(r):
        for col in range(H // 16):
            rows0[r, pl.ds(col * 16, 16)] = z16

    @pl.loop(0, NZF)
    def _(j):
        pltpu.sync_copy(rows0, acc_sh.at[pl.ds(s * NPT + j * CK, CK)])

    pltpu.sync_copy(rows0.at[pl.ds(0, NZT)],
                    acc_sh.at[pl.ds(s * NPT + NZF * CK, NZT)])
    plsc.subcore_barrier()

    def _scale_rows(rows, base):
        @pl.loop(0, CK)
        def _(r):
            idx16 = jnp.zeros((16,), jnp.int32) + (base + r)
            av = plsc.load_gather(sc_v, [idx16])
            for col in range(H // 16):
                sl = pl.ds(col * 16, 16)
                rows[r, sl] = rows[r, sl] * av

    def _gather(k, rows, sem):
        return pltpu.async_copy(wh_hbm.at[src2_v.at[k]], rows, sem)

    _gather(0, rows0, sem0)

    @pl.loop(0, RPW, step=2)
    def _(k):
        _gather(k + 1, rows1, sem1)
        pltpu.make_async_copy(wh_hbm.at[src2_v.at[0]], rows0, sem0).wait()
        _scale_rows(rows0, k * CK)
        pltpu.sync_copy(rows0, acc_sh.at[tgt2_v.at[k]], add=True)

        @pl.when(k + 2 < RPW)
        def _():
            _gather(k + 2, rows0, sem0)

        pltpu.make_async_copy(wh_hbm.at[src2_v.at[0]], rows1, sem1).wait()
        _scale_rows(rows1, (k + 1) * CK)
        pltpu.sync_copy(rows1, acc_sh.at[tgt2_v.at[k + 1]], add=True)

    plsc.subcore_barrier()

    out_base = c * N + s * NPT

    @pl.loop(0, NZF)
    def _(j):
        pltpu.sync_copy(acc_sh.at[pl.ds(s * NPT + j * CK, CK)], rows0)
        pltpu.sync_copy(rows0, out_hbm.at[pl.ds(out_base + j * CK, CK)])

    pltpu.sync_copy(acc_sh.at[pl.ds(s * NPT + NZF * CK, NZT)],
                    rows0.at[pl.ds(0, NZT)])
    pltpu.sync_copy(rows0.at[pl.ds(0, NZT)],
                    out_hbm.at[pl.ds(out_base + NZF * CK, NZT)])


_sc_scatter = functools.partial(
    pl.kernel,
    out_type=jax.ShapeDtypeStruct((NC * N, H), jnp.float32),
    mesh=_mesh,
    compiler_params=pltpu.CompilerParams(
        needs_layout_passes=False, use_tc_tiling_on_sc=False),
    scratch_types=[
        pltpu.VMEM((RPW, CK), jnp.int32),
        pltpu.VMEM((RPW, CK), jnp.int32),
        pltpu.VMEM((EPP,), jnp.float32),
        pltpu.VMEM((CK, H), jnp.float32),
        pltpu.VMEM((CK, H), jnp.float32),
        pltpu.VMEM_SHARED((N, H), jnp.float32),
        pltpu.SemaphoreType.DMA,
        pltpu.SemaphoreType.DMA,
    ],
)(_sc_scatter_body)


def _sc_gather_body(src_hbm, tgt_hbm, a_hbm, b_hbm, out_hbm,
                    src2_v, tgt2_v, ra0, rb0, ra1, rb1,
                    sa0, sb0, sa1, sb1):
    w = _worker()
    pltpu.sync_copy(src_hbm.at[pl.ds(w * RPW, RPW)], src2_v)
    pltpu.sync_copy(tgt_hbm.at[pl.ds(w * RPW, RPW)], tgt2_v)

    def _add_store(ra, rb, k):
        @pl.loop(0, CK)
        def _(r):
            for col in range(H // 16):
                sl = pl.ds(col * 16, 16)
                ra[r, sl] = ra[r, sl] + rb[r, sl]

        pltpu.sync_copy(ra, out_hbm.at[pl.ds(w * EPP + k * CK, CK)])

    def _gather(k, ra, rb, sa, sb):
        pltpu.async_copy(a_hbm.at[src2_v.at[k]], ra, sa)
        pltpu.async_copy(b_hbm.at[tgt2_v.at[k]], rb, sb)

    def _wait(ra, rb, sa, sb):
        pltpu.make_async_copy(a_hbm.at[src2_v.at[0]], ra, sa).wait()
        pltpu.make_async_copy(b_hbm.at[tgt2_v.at[0]], rb, sb).wait()

    _gather(0, ra0, rb0, sa0, sb0)

    @pl.loop(0, RPW, step=2)
    def _(k):
        _gather(k + 1, ra1, rb1, sa1, sb1)
        _wait(ra0, rb0, sa0, sb0)
        _add_store(ra0, rb0, k)

        @pl.when(k + 2 < RPW)
        def _():
            _gather(k + 2, ra0, rb0, sa0, sb0)

        _wait(ra1, rb1, sa1, sb1)
        _add_store(ra1, rb1, k + 1)


_sc_gather = functools.partial(
    pl.kernel,
    out_type=jax.ShapeDtypeStruct((E_PAD, H), jnp.float32),
    mesh=_mesh,
    compiler_params=pltpu.CompilerParams(
        needs_layout_passes=False, use_tc_tiling_on_sc=False),
    scratch_types=[
        pltpu.VMEM((RPW, CK), jnp.int32),
        pltpu.VMEM((RPW, CK), jnp.int32),
        pltpu.VMEM((CK, H), jnp.float32),
        pltpu.VMEM((CK, H), jnp.float32),
        pltpu.VMEM((CK, H), jnp.float32),
        pltpu.VMEM((CK, H), jnp.float32),
        pltpu.SemaphoreType.DMA,
        pltpu.SemaphoreType.DMA,
        pltpu.SemaphoreType.DMA,
        pltpu.SemaphoreType.DMA,
    ],
)(_sc_gather_body)


def _mm(x, P, q):
    n, k = x.shape
    m = P.shape[1]
    bn = 2000

    def body(x_ref, p_ref, q_ref, o_ref):
        o_ref[...] = (jnp.dot(x_ref[...], p_ref[...],
                              preferred_element_type=jnp.float32) + q_ref[...])

    return pl.pallas_call(
        body,
        grid=(n // bn,),
        in_specs=[
            pl.BlockSpec((bn, k), lambda i: (i, 0)),
            pl.BlockSpec((k, m), lambda i: (0, 0)),
            pl.BlockSpec((1, m), lambda i: (0, 0)),
        ],
        out_specs=pl.BlockSpec((bn, m), lambda i: (i, 0)),
        out_shape=jax.ShapeDtypeStruct((n, m), jnp.float32),
    )(x, P, q)


def _fuse(m0, m1, P, q):
    n = m0.shape[0]
    m = P.shape[1]
    bn = 2000

    def body(a_ref, b_ref, p_ref, q_ref, o_ref):
        h = a_ref[...] + b_ref[...]
        h = jnp.where(h >= 0, h, 0.2 * h)
        o_ref[...] = (jnp.dot(h, p_ref[...],
                              preferred_element_type=jnp.float32) + q_ref[...])

    return pl.pallas_call(
        body,
        grid=(n // bn,),
        in_specs=[
            pl.BlockSpec((bn, H), lambda i: (i, 0)),
            pl.BlockSpec((bn, H), lambda i: (i, 0)),
            pl.BlockSpec((H, m), lambda i: (0, 0)),
            pl.BlockSpec((1, m), lambda i: (0, 0)),
        ],
        out_specs=pl.BlockSpec((bn, m), lambda i: (i, 0)),
        out_shape=jax.ShapeDtypeStruct((n, m), jnp.float32),
    )(m0, m1, P, q)


def _head(G, ea, wc, b1, w2, b2, w3, b3):
    ne = G.shape[0]
    be = 4096

    def body(g_ref, e_ref, wc_ref, b1_ref, w2_ref, b2_ref, w3_ref, b3_ref,
             o_ref):
        x1 = (g_ref[...] +
              jnp.dot(e_ref[...], wc_ref[...],
                      preferred_element_type=jnp.float32) + b1_ref[...])
        x1 = jnp.maximum(x1, 0.0)
        x2 = jnp.maximum(
            jnp.dot(x1, w2_ref[...], preferred_element_type=jnp.float32)
            + b2_ref[...], 0.0)
        o_ref[...] = (jnp.dot(x2, w3_ref[...],
                              preferred_element_type=jnp.float32) + b3_ref[...])

    return pl.pallas_call(
        body,
        grid=(ne // be,),
        in_specs=[
            pl.BlockSpec((be, H), lambda i: (i, 0)),
            pl.BlockSpec((be, DE), lambda i: (i, 0)),
            pl.BlockSpec((DE, H), lambda i: (0, 0)),
            pl.BlockSpec((1, H), lambda i: (0, 0)),
            pl.BlockSpec((H, 32), lambda i: (0, 0)),
            pl.BlockSpec((1, 32), lambda i: (0, 0)),
            pl.BlockSpec((32, 1), lambda i: (0, 0)),
            pl.BlockSpec((1, 1), lambda i: (0, 0)),
        ],
        out_specs=pl.BlockSpec((be, 1), lambda i: (i, 0)),
        out_shape=jax.ShapeDtypeStruct((ne, 1), jnp.float32),
    )(G, ea, wc, b1, w2, b2, w3, b3)


def kernel(node_feats, edge_index, edge_attr, W1, bW1, a1, ba1,
           W2, bW2, a2, ba2, Hw1, Hb1, Hw2, Hb2, Hw3, Hb3):
    pad = ((0, 0), (0, EPP - EPW))
    src2d = jnp.pad(edge_index[:, 0].reshape(NW, EPW),
                    pad).reshape(E_PAD // CK, CK)
    tgt2d = jnp.pad(edge_index[:, 1].reshape(NW, EPW),
                    pad).reshape(E_PAD // CK, CK)
    ea_pad = jnp.pad(edge_attr.reshape(NW, EPW, DE),
                     ((0, 0), (0, EPP - EPW), (0, 0))).reshape(E_PAD, DE)

    def fold(Wl, bWl, al, bal):
        a_s = al[:H, 0]
        a_t = al[H:, 0]
        P = jnp.concatenate(
            [Wl, (Wl @ a_s)[:, None], (Wl @ a_t)[:, None]], axis=1)
        q = jnp.concatenate(
            [bWl, (bWl @ a_s + bal[0])[None], (bWl @ a_t)[None]])[None, :]
        return P, q

    def gat_edge_phase(Wh, asv, atv):
        scores = _sc_scores(src2d, tgt2d, asv, atv)
        wt = _tc_softmax(scores)
        msgs = _sc_scatter(src2d, tgt2d, wt, Wh)
        return msgs[:N], msgs[N:]

    P1, q1 = fold(W1, bW1, a1, ba1)
    o1 = _mm(node_feats, P1, q1)
    m0, m1 = gat_edge_phase(o1[:, :H], o1[:, H], o1[:, H + 1])

    P2, q2 = fold(W2, bW2, a2, ba2)
    o2 = _fuse(m0, m1, P2, q2)
    m0, m1 = gat_edge_phase(o2[:, :H], o2[:, H], o2[:, H + 1])

    P3 = jnp.concatenate([Hw1[:H], Hw1[H:2 * H]], axis=1)
    q3 = jnp.concatenate([jnp.zeros((H,), jnp.float32), Hb1])[None, :]
    o3 = _fuse(m0, m1, P3, q3)
    G = _sc_gather(src2d, tgt2d, o3[:, :H], o3[:, H:])
    y = _head(G, ea_pad, Hw1[2 * H:], jnp.zeros((1, H), jnp.float32),
              Hw2, Hb2[None, :], Hw3, Hb3[None, :])
    return y[:, 0].reshape(NW, EPP)[:, :EPW].reshape(E)
